# baseline (speedup 1.0000x reference)
import jax
import jax.numpy as jnp
from jax import lax
from jax.experimental import pallas as pl
from jax.experimental.pallas import tpu as pltpu
from jax.experimental.pallas import tpu_sc as plsc

_N, _E, _D, _O = 10000, 320000, 128, 300
_DH = _D // 2
_NC, _NS = 2, 16
_EPT = _E // _NS
_K = 80
_NCH = _EPT // _K
_NPAD = 10240
_RPT = _NPAD // _NS
_DW = 8
_NB = 5
_BN = 1000

_MESH = plsc.VectorSubcoreMesh(
    core_axis_name="c", subcore_axis_name="s", num_cores=_NC, num_subcores=_NS
)


def _make_sc_pass(with_deg):
  out_type = [jax.ShapeDtypeStruct((_NPAD, _D), jnp.float32)]
  if with_deg:
    out_type.append(jax.ShapeDtypeStruct((_NC, _NPAD, _DW), jnp.float32))

  scratch = [
      pltpu.VMEM((_EPT,), jnp.int32),
      pltpu.VMEM((_EPT,), jnp.int32),
      [pltpu.VMEM((_K, _DH), jnp.float32)] * _NB,
      pltpu.VMEM((_K, _DW), jnp.float32),
      pltpu.VMEM_SHARED((_NPAD, _DH), jnp.float32),
      pltpu.VMEM_SHARED((_NPAD, _DW), jnp.float32),
      [pltpu.SemaphoreType.DMA] * _NB,
      [pltpu.SemaphoreType.DMA] * _NB,
      [pltpu.SemaphoreType.DMA] * _NB,
  ]

  def body(feat2_hbm, src_hbm, dst_hbm, zrow_hbm, zdeg_hbm, onesk_hbm, *rest):
    if with_deg:
      part_hbm, degp_hbm = rest[0], rest[1]
      srcv, dstv, rowsb, onesv, acc, dacc, sems, ssems, dsems = rest[2:]
    else:
      part_hbm = rest[0]
      srcv, dstv, rowsb, onesv, acc, dacc, sems, ssems, dsems = rest[1:]
    c = lax.axis_index("c")
    s = lax.axis_index("s")
    feat = feat2_hbm.at[c]

    pltpu.sync_copy(src_hbm.at[pl.ds(s * _EPT, _EPT)], srcv)
    pltpu.sync_copy(dst_hbm.at[pl.ds(s * _EPT, _EPT)], dstv)
    pltpu.sync_copy(onesk_hbm, onesv)
    pltpu.sync_copy(zrow_hbm, acc.at[pl.ds(s * _RPT, _RPT)])
    if with_deg:
      pltpu.sync_copy(zdeg_hbm, dacc.at[pl.ds(s * _RPT, _RPT)])

    for b in range(_NB):
      pltpu.async_copy(feat.at[srcv.at[pl.ds(b * _K, _K)]], rowsb[b], sems[b])
    plsc.subcore_barrier()

    def group(g, carry):
      for b in range(_NB):
        j = g * _NB + b
        src_idx = srcv.at[pl.ds(j * _K, _K)]
        dst_idx = dstv.at[pl.ds(j * _K, _K)]
        pltpu.make_async_copy(feat.at[src_idx], rowsb[b], sems[b]).wait()
        pltpu.async_copy(rowsb[b], acc.at[dst_idx], ssems[b], add=True)
        if with_deg:
          pltpu.async_copy(onesv, dacc.at[dst_idx], dsems[b], add=True)
      for b in range(_NB):
        j = g * _NB + b
        dst_idx = dstv.at[pl.ds(j * _K, _K)]
        pltpu.make_async_copy(rowsb[b], acc.at[dst_idx], ssems[b]).wait()
        if with_deg:
          pltpu.make_async_copy(onesv, dacc.at[dst_idx], dsems[b]).wait()

        @pl.when(j + _NB < _NCH)
        def _():
          nxt = srcv.at[pl.ds((j + _NB) * _K, _K)]
          pltpu.async_copy(feat.at[nxt], rowsb[b], sems[b])
      return carry

    lax.fori_loop(0, _NCH // _NB, group, 0)
    plsc.subcore_barrier()

    pltpu.sync_copy(acc.at[pl.ds(s * _RPT, _RPT)],
                    part_hbm.at[pl.ds(s * _RPT, _RPT), pl.ds(c * _DH, _DH)])
    if with_deg:
      pltpu.sync_copy(dacc.at[pl.ds(s * _RPT, _RPT)],
                      degp_hbm.at[c, pl.ds(s * _RPT, _RPT)])

  return pl.kernel(
      body, out_type=tuple(out_type), mesh=_MESH, scratch_types=tuple(scratch),
      compiler_params=pltpu.CompilerParams(use_tc_tiling_on_sc=False),
  )


_sc_pass1 = _make_sc_pass(True)
_sc_pass2 = _make_sc_pass(False)


def _tc1_body(part_ref, degp_ref, x_ref, wl_ref, b_ref, wr_ref, h_ref, e_ref):
  deg = degp_ref[0][:, 0:1]
  mean = part_ref[...] / jnp.maximum(deg, 1.0)
  h = (jnp.dot(mean, wl_ref[...], preferred_element_type=jnp.float32)
       + b_ref[...]
       + jnp.dot(x_ref[...], wr_ref[...], preferred_element_type=jnp.float32))
  h_ref[...] = h
  xemb = jnp.maximum(h, 0.0)
  e_ref[0] = xemb[:, :_DH]
  e_ref[1] = xemb[:, _DH:]


def _tc2_body(part_ref, degp_ref, h_ref, wl_ref, b_ref, wr_ref, o_ref):
  deg = degp_ref[0][:, 0:1]
  mean = part_ref[...] / jnp.maximum(deg, 1.0)
  xemb = jnp.maximum(h_ref[...], 0.0)
  logits = (jnp.dot(mean, wl_ref[...], preferred_element_type=jnp.float32)
            + b_ref[...]
            + jnp.dot(xemb, wr_ref[...], preferred_element_type=jnp.float32))
  m = jnp.max(logits, axis=1, keepdims=True)
  lse = m + jnp.log(jnp.sum(jnp.exp(logits - m), axis=1, keepdims=True))
  o_ref[...] = logits - lse


def _tc1(part, degp, x, wl, b, wr):
  return pl.pallas_call(
      _tc1_body,
      grid=(_N // _BN,),
      in_specs=[
          pl.BlockSpec((_BN, _D), lambda i: (i, 0)),
          pl.BlockSpec((_NC, _BN, _DW), lambda i: (0, i, 0)),
          pl.BlockSpec((_BN, _D), lambda i: (i, 0)),
          pl.BlockSpec((_D, _D), lambda i: (0, 0)),
          pl.BlockSpec((1, _D), lambda i: (0, 0)),
          pl.BlockSpec((_D, _D), lambda i: (0, 0)),
      ],
      out_specs=[
          pl.BlockSpec((_BN, _D), lambda i: (i, 0)),
          pl.BlockSpec((_NC, _BN, _DH), lambda i: (0, i, 0)),
      ],
      out_shape=[
          jax.ShapeDtypeStruct((_N, _D), jnp.float32),
          jax.ShapeDtypeStruct((_NC, _N, _DH), jnp.float32),
      ],
  )(part, degp, x, wl, b, wr)


def _tc2(part, degp, h, wl, b, wr):
  return pl.pallas_call(
      _tc2_body,
      grid=(_N // _BN,),
      in_specs=[
          pl.BlockSpec((_BN, _D), lambda i: (i, 0)),
          pl.BlockSpec((_NC, _BN, _DW), lambda i: (0, i, 0)),
          pl.BlockSpec((_BN, _D), lambda i: (i, 0)),
          pl.BlockSpec((_D, _O), lambda i: (0, 0)),
          pl.BlockSpec((1, _O), lambda i: (0, 0)),
          pl.BlockSpec((_D, _O), lambda i: (0, 0)),
      ],
      out_specs=pl.BlockSpec((_BN, _O), lambda i: (i, 0)),
      out_shape=jax.ShapeDtypeStruct((_N, _O), jnp.float32),
  )(part, degp, h, wl, b, wr)


def kernel(x, edge_index, W_l1, b_l1, W_r1, W_l2, b_l2, W_r2):
  src = edge_index[0]
  dst = edge_index[1]
  x2 = jnp.stack([x[:, :_DH], x[:, _DH:]])
  zrow = jnp.zeros((_RPT, _DH), jnp.float32)
  zdeg = jnp.zeros((_RPT, _DW), jnp.float32)
  onesk = jnp.ones((_K, _DW), jnp.float32)

  part1, degp = _sc_pass1(x2, src, dst, zrow, zdeg, onesk)
  h, xemb2 = _tc1(part1, degp, x, W_l1, b_l1.reshape(1, _D), W_r1)
  part2 = _sc_pass2(xemb2, src, dst, zrow, zdeg, onesk)
  if isinstance(part2, (tuple, list)):
    part2 = part2[0]
  out = _tc2(part2, degp, h, W_l2, b_l2.reshape(1, _O), W_r2)
  return (h, out)


# speedup vs baseline: 1.1074x; 1.1074x over previous
import jax
import jax.numpy as jnp
from jax import lax
from jax.experimental import pallas as pl
from jax.experimental.pallas import tpu as pltpu
from jax.experimental.pallas import tpu_sc as plsc

_N, _E, _D, _O = 10000, 320000, 128, 300
_DH = _D // 2
_NC, _NS = 2, 16
_EPT = _E // _NS
_K = 80
_NCH = _EPT // _K
_NPAD = 10240
_RPT = _NPAD // _NS
_DW = 8
_NB = 5
_BN = 1000

_MESH = plsc.VectorSubcoreMesh(
    core_axis_name="c", subcore_axis_name="s", num_cores=_NC, num_subcores=_NS
)


def _make_sc_pass(with_deg):
  out_type = [jax.ShapeDtypeStruct((_NPAD, _D), jnp.float32)]
  if with_deg:
    out_type.append(jax.ShapeDtypeStruct((_NC, _NPAD, _DW), jnp.float32))

  scratch = [
      pltpu.VMEM((_EPT,), jnp.int32),
      pltpu.VMEM((_EPT,), jnp.int32),
      [pltpu.VMEM((_K, _DH), jnp.float32)] * _NB,
      pltpu.VMEM((_K, _DW), jnp.float32),
      pltpu.VMEM_SHARED((_NPAD, _DH), jnp.float32),
      pltpu.VMEM_SHARED((_NPAD, _DW), jnp.float32),
      [pltpu.SemaphoreType.DMA] * _NB,
      pltpu.SemaphoreType.DMA,
  ]

  def body(feat2_hbm, src_hbm, dst_hbm, zrow_hbm, zdeg_hbm, onesk_hbm, *rest):
    if with_deg:
      part_hbm, degp_hbm = rest[0], rest[1]
      srcv, dstv, rowsb, onesv, acc, dacc, sems, dsem = rest[2:]
    else:
      part_hbm = rest[0]
      srcv, dstv, rowsb, onesv, acc, dacc, sems, dsem = rest[1:]
    c = lax.axis_index("c")
    s = lax.axis_index("s")
    feat = feat2_hbm.at[c]

    pltpu.sync_copy(src_hbm.at[pl.ds(s * _EPT, _EPT)], srcv)
    pltpu.sync_copy(dst_hbm.at[pl.ds(s * _EPT, _EPT)], dstv)
    pltpu.sync_copy(onesk_hbm, onesv)
    pltpu.sync_copy(zrow_hbm, acc.at[pl.ds(s * _RPT, _RPT)])
    if with_deg:
      pltpu.sync_copy(zdeg_hbm, dacc.at[pl.ds(s * _RPT, _RPT)])

    for b in range(_NB):
      pltpu.async_copy(feat.at[srcv.at[pl.ds(b * _K, _K)]], rowsb[b], sems[b])
    plsc.subcore_barrier()

    def group(g, carry):
      for b in range(_NB):
        j = g * _NB + b
        src_idx = srcv.at[pl.ds(j * _K, _K)]
        dst_idx = dstv.at[pl.ds(j * _K, _K)]
        pltpu.make_async_copy(feat.at[src_idx], rowsb[b], sems[b]).wait()
        if with_deg:
          @pl.when(j >= _NB)
          def _():
            prv = dstv.at[pl.ds((j - _NB) * _K, _K)]
            pltpu.make_async_copy(onesv, dacc.at[prv], dsem).wait()
          pltpu.async_copy(onesv, dacc.at[dst_idx], dsem, add=True)
        pltpu.sync_copy(rowsb[b], acc.at[dst_idx], add=True)

        @pl.when(j + _NB < _NCH)
        def _():
          nxt = srcv.at[pl.ds((j + _NB) * _K, _K)]
          pltpu.async_copy(feat.at[nxt], rowsb[b], sems[b])
      return carry

    lax.fori_loop(0, _NCH // _NB, group, 0)
    if with_deg:
      for b in range(_NB):
        prv = dstv.at[pl.ds((_NCH - _NB + b) * _K, _K)]
        pltpu.make_async_copy(onesv, dacc.at[prv], dsem).wait()
    plsc.subcore_barrier()

    pltpu.sync_copy(acc.at[pl.ds(s * _RPT, _RPT)],
                    part_hbm.at[pl.ds(s * _RPT, _RPT), pl.ds(c * _DH, _DH)])
    if with_deg:
      pltpu.sync_copy(dacc.at[pl.ds(s * _RPT, _RPT)],
                      degp_hbm.at[c, pl.ds(s * _RPT, _RPT)])

  return pl.kernel(
      body, out_type=tuple(out_type), mesh=_MESH, scratch_types=tuple(scratch),
      compiler_params=pltpu.CompilerParams(use_tc_tiling_on_sc=False),
  )


_sc_pass1 = _make_sc_pass(True)
_sc_pass2 = _make_sc_pass(False)


def _tc1_body(part_ref, degp_ref, x_ref, wl_ref, b_ref, wr_ref, h_ref, e_ref):
  deg = degp_ref[0][:, 0:1]
  mean = part_ref[...] / jnp.maximum(deg, 1.0)
  h = (jnp.dot(mean, wl_ref[...], preferred_element_type=jnp.float32)
       + b_ref[...]
       + jnp.dot(x_ref[...], wr_ref[...], preferred_element_type=jnp.float32))
  h_ref[...] = h
  xemb = jnp.maximum(h, 0.0)
  e_ref[0] = xemb[:, :_DH]
  e_ref[1] = xemb[:, _DH:]


def _tc2_body(part_ref, degp_ref, h_ref, wl_ref, b_ref, wr_ref, o_ref):
  deg = degp_ref[0][:, 0:1]
  mean = part_ref[...] / jnp.maximum(deg, 1.0)
  xemb = jnp.maximum(h_ref[...], 0.0)
  logits = (jnp.dot(mean, wl_ref[...], preferred_element_type=jnp.float32)
            + b_ref[...]
            + jnp.dot(xemb, wr_ref[...], preferred_element_type=jnp.float32))
  m = jnp.max(logits, axis=1, keepdims=True)
  lse = m + jnp.log(jnp.sum(jnp.exp(logits - m), axis=1, keepdims=True))
  o_ref[...] = logits - lse


def _tc1(part, degp, x, wl, b, wr):
  return pl.pallas_call(
      _tc1_body,
      grid=(_N // _BN,),
      in_specs=[
          pl.BlockSpec((_BN, _D), lambda i: (i, 0)),
          pl.BlockSpec((_NC, _BN, _DW), lambda i: (0, i, 0)),
          pl.BlockSpec((_BN, _D), lambda i: (i, 0)),
          pl.BlockSpec((_D, _D), lambda i: (0, 0)),
          pl.BlockSpec((1, _D), lambda i: (0, 0)),
          pl.BlockSpec((_D, _D), lambda i: (0, 0)),
      ],
      out_specs=[
          pl.BlockSpec((_BN, _D), lambda i: (i, 0)),
          pl.BlockSpec((_NC, _BN, _DH), lambda i: (0, i, 0)),
      ],
      out_shape=[
          jax.ShapeDtypeStruct((_N, _D), jnp.float32),
          jax.ShapeDtypeStruct((_NC, _N, _DH), jnp.float32),
      ],
  )(part, degp, x, wl, b, wr)


def _tc2(part, degp, h, wl, b, wr):
  return pl.pallas_call(
      _tc2_body,
      grid=(_N // _BN,),
      in_specs=[
          pl.BlockSpec((_BN, _D), lambda i: (i, 0)),
          pl.BlockSpec((_NC, _BN, _DW), lambda i: (0, i, 0)),
          pl.BlockSpec((_BN, _D), lambda i: (i, 0)),
          pl.BlockSpec((_D, _O), lambda i: (0, 0)),
          pl.BlockSpec((1, _O), lambda i: (0, 0)),
          pl.BlockSpec((_D, _O), lambda i: (0, 0)),
      ],
      out_specs=pl.BlockSpec((_BN, _O), lambda i: (i, 0)),
      out_shape=jax.ShapeDtypeStruct((_N, _O), jnp.float32),
  )(part, degp, h, wl, b, wr)


def kernel(x, edge_index, W_l1, b_l1, W_r1, W_l2, b_l2, W_r2):
  src = edge_index[0]
  dst = edge_index[1]
  x2 = jnp.stack([x[:, :_DH], x[:, _DH:]])
  zrow = jnp.zeros((_RPT, _DH), jnp.float32)
  zdeg = jnp.zeros((_RPT, _DW), jnp.float32)
  onesk = jnp.ones((_K, _DW), jnp.float32)

  part1, degp = _sc_pass1(x2, src, dst, zrow, zdeg, onesk)
  h, xemb2 = _tc1(part1, degp, x, W_l1, b_l1.reshape(1, _D), W_r1)
  part2 = _sc_pass2(xemb2, src, dst, zrow, zdeg, onesk)
  if isinstance(part2, (tuple, list)):
    part2 = part2[0]
  out = _tc2(part2, degp, h, W_l2, b_l2.reshape(1, _O), W_r2)
  return (h, out)


# speedup vs baseline: 1.2468x; 1.1259x over previous
import jax
import jax.numpy as jnp
from jax import lax
from jax.experimental import pallas as pl
from jax.experimental.pallas import tpu as pltpu
from jax.experimental.pallas import tpu_sc as plsc

_N, _E, _D, _O = 10000, 320000, 128, 300
_DH = _D // 2
_NC, _NS = 2, 16
_EPT = _E // _NS
_K = 80
_NCH = _EPT // _K
_NPAD = 10240
_RPT = _NPAD // _NS
_DW = 8
_NB = 5
_BN = 1000

_MESH = plsc.VectorSubcoreMesh(
    core_axis_name="c", subcore_axis_name="s", num_cores=_NC, num_subcores=_NS
)


def _make_sc_pass(with_deg):
  out_type = [jax.ShapeDtypeStruct((_NPAD, _D), jnp.float32)]
  if with_deg:
    out_type.append(jax.ShapeDtypeStruct((_NC, _NPAD, _DW), jnp.float32))

  scratch = [
      pltpu.VMEM((_EPT,), jnp.int32),
      pltpu.VMEM((_EPT,), jnp.int32),
      [pltpu.VMEM((_K,), jnp.int32)] * _NB,
      [pltpu.VMEM((_K, _DH), jnp.float32)] * _NB,
      pltpu.VMEM((_K, _DW), jnp.float32),
      pltpu.VMEM_SHARED((_NPAD, _DH), jnp.float32),
      pltpu.VMEM_SHARED((_NPAD, _DW), jnp.float32),
      [pltpu.SemaphoreType.DMA] * _NB,
      pltpu.SemaphoreType.DMA,
  ]

  def body(feat_hbm, edge_hbm, zrow_hbm, zdeg_hbm, onesk_hbm, *rest):
    if with_deg:
      part_hbm, degp_hbm = rest[0], rest[1]
      srcv, dstv, idxb, rowsb, onesv, acc, dacc, sems, dsem = rest[2:]
    else:
      part_hbm = rest[0]
      srcv, dstv, idxb, rowsb, onesv, acc, dacc, sems, dsem = rest[1:]
    c = lax.axis_index("c")
    s = lax.axis_index("s")
    feat = feat_hbm

    def xform(jj, b):
      for g in range(_K // 16):
        v = srcv[pl.ds(jj * _K + g * 16, 16)]
        idxb[b][pl.ds(g * 16, 16)] = v * 2 + c

    pltpu.sync_copy(edge_hbm.at[0, pl.ds(s * _EPT, _EPT)], srcv)
    pltpu.sync_copy(edge_hbm.at[1, pl.ds(s * _EPT, _EPT)], dstv)
    pltpu.sync_copy(onesk_hbm, onesv)
    pltpu.sync_copy(zrow_hbm, acc.at[pl.ds(s * _RPT, _RPT)])
    if with_deg:
      pltpu.sync_copy(zdeg_hbm, dacc.at[pl.ds(s * _RPT, _RPT)])

    for b in range(_NB):
      xform(b, b)
      pltpu.async_copy(feat.at[idxb[b]], rowsb[b], sems[b])
    plsc.subcore_barrier()

    def group(g, carry):
      for b in range(_NB):
        j = g * _NB + b
        dst_idx = dstv.at[pl.ds(j * _K, _K)]
        pltpu.make_async_copy(feat.at[idxb[b]], rowsb[b], sems[b]).wait()
        if with_deg:
          @pl.when(j >= _NB)
          def _():
            prv = dstv.at[pl.ds((j - _NB) * _K, _K)]
            pltpu.make_async_copy(onesv, dacc.at[prv], dsem).wait()
          pltpu.async_copy(onesv, dacc.at[dst_idx], dsem, add=True)
        pltpu.sync_copy(rowsb[b], acc.at[dst_idx], add=True)

        @pl.when(j + _NB < _NCH)
        def _():
          xform(j + _NB, b)
          pltpu.async_copy(feat.at[idxb[b]], rowsb[b], sems[b])
      return carry

    lax.fori_loop(0, _NCH // _NB, group, 0)
    if with_deg:
      for b in range(_NB):
        prv = dstv.at[pl.ds((_NCH - _NB + b) * _K, _K)]
        pltpu.make_async_copy(onesv, dacc.at[prv], dsem).wait()
    plsc.subcore_barrier()

    pltpu.sync_copy(acc.at[pl.ds(s * _RPT, _RPT)],
                    part_hbm.at[pl.ds(s * _RPT, _RPT), pl.ds(c * _DH, _DH)])
    if with_deg:
      pltpu.sync_copy(dacc.at[pl.ds(s * _RPT, _RPT)],
                      degp_hbm.at[c, pl.ds(s * _RPT, _RPT)])

  return pl.kernel(
      body, out_type=tuple(out_type), mesh=_MESH, scratch_types=tuple(scratch),
      compiler_params=pltpu.CompilerParams(use_tc_tiling_on_sc=False),
  )


_sc_pass1 = _make_sc_pass(True)
_sc_pass2 = _make_sc_pass(False)


def _tc1_body(part_ref, degp_ref, x_ref, wl_ref, b_ref, wr_ref, h_ref, e_ref):
  deg = degp_ref[0][:, 0:1]
  mean = part_ref[...] / jnp.maximum(deg, 1.0)
  h = (jnp.dot(mean, wl_ref[...], preferred_element_type=jnp.float32)
       + b_ref[...]
       + jnp.dot(x_ref[...], wr_ref[...], preferred_element_type=jnp.float32))
  h_ref[...] = h
  e_ref[...] = jnp.maximum(h, 0.0)


def _tc2_body(part_ref, degp_ref, h_ref, wl_ref, b_ref, wr_ref, o_ref):
  deg = degp_ref[0][:, 0:1]
  mean = part_ref[...] / jnp.maximum(deg, 1.0)
  xemb = jnp.maximum(h_ref[...], 0.0)
  logits = (jnp.dot(mean, wl_ref[...], preferred_element_type=jnp.float32)
            + b_ref[...]
            + jnp.dot(xemb, wr_ref[...], preferred_element_type=jnp.float32))
  m = jnp.max(logits, axis=1, keepdims=True)
  lse = m + jnp.log(jnp.sum(jnp.exp(logits - m), axis=1, keepdims=True))
  o_ref[...] = logits - lse


def _tc1(part, degp, x, wl, b, wr):
  return pl.pallas_call(
      _tc1_body,
      grid=(_N // _BN,),
      in_specs=[
          pl.BlockSpec((_BN, _D), lambda i: (i, 0)),
          pl.BlockSpec((_NC, _BN, _DW), lambda i: (0, i, 0)),
          pl.BlockSpec((_BN, _D), lambda i: (i, 0)),
          pl.BlockSpec((_D, _D), lambda i: (0, 0)),
          pl.BlockSpec((1, _D), lambda i: (0, 0)),
          pl.BlockSpec((_D, _D), lambda i: (0, 0)),
      ],
      out_specs=[
          pl.BlockSpec((_BN, _D), lambda i: (i, 0)),
          pl.BlockSpec((_BN, _D), lambda i: (i, 0)),
      ],
      out_shape=[
          jax.ShapeDtypeStruct((_N, _D), jnp.float32),
          jax.ShapeDtypeStruct((_N, _D), jnp.float32),
      ],
  )(part, degp, x, wl, b, wr)


def _tc2(part, degp, h, wl, b, wr):
  return pl.pallas_call(
      _tc2_body,
      grid=(_N // _BN,),
      in_specs=[
          pl.BlockSpec((_BN, _D), lambda i: (i, 0)),
          pl.BlockSpec((_NC, _BN, _DW), lambda i: (0, i, 0)),
          pl.BlockSpec((_BN, _D), lambda i: (i, 0)),
          pl.BlockSpec((_D, _O), lambda i: (0, 0)),
          pl.BlockSpec((1, _O), lambda i: (0, 0)),
          pl.BlockSpec((_D, _O), lambda i: (0, 0)),
      ],
      out_specs=pl.BlockSpec((_BN, _O), lambda i: (i, 0)),
      out_shape=jax.ShapeDtypeStruct((_N, _O), jnp.float32),
  )(part, degp, h, wl, b, wr)


def kernel(x, edge_index, W_l1, b_l1, W_r1, W_l2, b_l2, W_r2):
  zrow = jnp.zeros((_RPT, _DH), jnp.float32)
  zdeg = jnp.zeros((_RPT, _DW), jnp.float32)
  onesk = jnp.ones((_K, _DW), jnp.float32)

  part1, degp = _sc_pass1(x.reshape(2 * _N, _DH), edge_index,
                          zrow, zdeg, onesk)
  h, xemb = _tc1(part1, degp, x, W_l1, b_l1.reshape(1, _D), W_r1)
  part2 = _sc_pass2(xemb.reshape(2 * _N, _DH), edge_index,
                    zrow, zdeg, onesk)
  if isinstance(part2, (tuple, list)):
    part2 = part2[0]
  out = _tc2(part2, degp, h, W_l2, b_l2.reshape(1, _O), W_r2)
  return (h, out)
